# baseline (speedup 1.0000x reference)
import jax
import jax.numpy as jnp
from jax import lax
from jax.experimental import pallas as pl
from jax.experimental.pallas import tpu as pltpu
from jax.experimental.pallas import tpu_sc as plsc

N = 1024
U = 200
D = 64
VECS = D // 16

_info = plsc.get_sparse_core_info()
NC, NS = _info.num_cores, _info.num_subcores
NW = NC * NS
SEQ_PER_W = N // NW
PAIRS_PER_W = SEQ_PER_W // 2

CH0 = 128
CH1 = U - CH0

V = 1000000
TB = 512
TGRID = (V + TB - 1) // TB


def _tc_repack_body(tT_ref, out_ref):
    x = tT_ref[...]
    xt = x.T
    out_ref[...] = jnp.concatenate([xt[0:TB // 2], xt[TB // 2:TB]], axis=1)


_tc_repack = pl.pallas_call(
    _tc_repack_body,
    out_shape=jax.ShapeDtypeStruct((TGRID * (TB // 2), 128), jnp.float32),
    grid=(TGRID,),
    in_specs=[pl.BlockSpec((64, TB), lambda i: (0, i))],
    out_specs=pl.BlockSpec((TB // 2, 128), lambda i: (i, 0)),
)


def _sc_body(y2_hbm, half_hbm, table_hbm, w_hbm, out_hbm,
             idx_v, half_v, rows_v, outb_v, w_v, sem):
    wid = lax.axis_index("s") * NC + lax.axis_index("c")

    pltpu.sync_copy(w_hbm, w_v)
    w0 = [w_v[0, pl.ds(16 * j, 16)] for j in range(VECS)]
    w1 = [w_v[1, pl.ds(16 * j, 16)] for j in range(VECS)]
    zero = jnp.zeros((16,), jnp.float32)

    def pair_body(p_i, carry):
        pair0 = wid * PAIRS_PER_W + p_i
        pair_base = pair0 * 2 * U

        def do_seq(si):
            base = pair_base + si * U
            pltpu.sync_copy(y2_hbm.at[pl.ds(base, U)], idx_v)
            pltpu.sync_copy(half_hbm.at[pl.ds(base, U)],
                            half_v.at[pl.ds(0, U)])
            cp0 = pltpu.async_copy(
                table_hbm.at[idx_v.at[pl.ds(0, CH0)]],
                rows_v.at[pl.ds(0, CH0)], sem)
            cp1 = pltpu.async_copy(
                table_hbm.at[idx_v.at[pl.ds(CH0, CH1)]],
                rows_v.at[pl.ds(CH0, CH1)], sem)
            cp0.wait()
            cp1.wait()

            def blk_body(blk, prev):
                rbase = 8 * blk
                hv = half_v[pl.ds(rbase, 16)] * D
                for t in range(8):
                    u = rbase + t
                    off = hv[t]
                    cur = tuple(
                        rows_v[u, pl.ds(off + 16 * j, 16)]
                        for j in range(VECS))
                    for j in range(VECS):
                        outb_v[si, u, pl.ds(16 * j, 16)] = jnp.maximum(
                            prev[j] * w0[j] + cur[j] * w1[j], 0.0)
                    prev = cur
                return prev

            lax.fori_loop(0, U // 8, blk_body, (zero,) * VECS)

        do_seq(0)
        do_seq(1)
        pltpu.sync_copy(outb_v, out_hbm.at[pl.ds(pair0 * 2, 2)])
        return carry

    lax.fori_loop(0, PAIRS_PER_W, pair_body, 0)


_sc_call = pl.kernel(
    _sc_body,
    out_type=jax.ShapeDtypeStruct((N, U, D), jnp.float32),
    mesh=plsc.VectorSubcoreMesh(core_axis_name="c", subcore_axis_name="s"),
    scratch_types=[
        pltpu.VMEM((U,), jnp.int32),
        pltpu.VMEM((U + 16,), jnp.int32),
        pltpu.VMEM((U, 2 * D), jnp.float32),
        pltpu.VMEM((2, U, D), jnp.float32),
        pltpu.VMEM((2, D), jnp.float32),
        pltpu.SemaphoreType.DMA,
    ],
)


@jax.jit
def kernel(y, table, conv_w):
    y_flat = y.reshape(N * U).astype(jnp.int32)
    y2 = ((y_flat >> 9) << 8) + (y_flat & 255)
    half = (y_flat >> 8) & 1
    t2 = _tc_repack(table.T)
    w = conv_w.T
    return _sc_call(y2, half, t2, w)


# speedup vs baseline: 2.0551x; 2.0551x over previous
import jax
import jax.numpy as jnp
from jax import lax
from jax.experimental import pallas as pl
from jax.experimental.pallas import tpu as pltpu
from jax.experimental.pallas import tpu_sc as plsc

N = 1024
U = 200
D = 64
VECS = D // 16

_info = plsc.get_sparse_core_info()
NC, NS = _info.num_cores, _info.num_subcores
NW = NC * NS
SEQ_PER_W = N // NW
PAIRS_PER_W = SEQ_PER_W // 2

CH0 = 128
CH1 = U - CH0

V = 1000000
TB = 2048
TGRID = (V + TB - 1) // TB


def _tc_repack_body(tT_ref, out_ref):
    x = tT_ref[...]
    xt = x.T
    out_ref[...] = jnp.concatenate([xt[0:TB // 2], xt[TB // 2:TB]], axis=1)


_tc_repack = pl.pallas_call(
    _tc_repack_body,
    out_shape=jax.ShapeDtypeStruct((TGRID * (TB // 2), 128), jnp.float32),
    grid=(TGRID,),
    in_specs=[pl.BlockSpec((64, TB), lambda i: (0, i))],
    out_specs=pl.BlockSpec((TB // 2, 128), lambda i: (i, 0)),
)


def _sc_body(y2_hbm, half_hbm, table_hbm, w_hbm, out_hbm,
             idx_v, half_v, rows_v, outb_v, w_v, sem):
    wid = lax.axis_index("s") * NC + lax.axis_index("c")

    pltpu.sync_copy(w_hbm, w_v)
    w0 = [w_v[0, pl.ds(16 * j, 16)] for j in range(VECS)]
    w1 = [w_v[1, pl.ds(16 * j, 16)] for j in range(VECS)]
    zero = jnp.zeros((16,), jnp.float32)

    def pair_body(p_i, carry):
        pair0 = wid * PAIRS_PER_W + p_i
        pair_base = pair0 * 2 * U

        def do_seq(si):
            base = pair_base + si * U
            pltpu.sync_copy(y2_hbm.at[pl.ds(base, U)], idx_v)
            pltpu.sync_copy(half_hbm.at[pl.ds(base, U)],
                            half_v.at[pl.ds(0, U)])
            cp0 = pltpu.async_copy(
                table_hbm.at[idx_v.at[pl.ds(0, CH0)]],
                rows_v.at[pl.ds(0, CH0)], sem)
            cp1 = pltpu.async_copy(
                table_hbm.at[idx_v.at[pl.ds(CH0, CH1)]],
                rows_v.at[pl.ds(CH0, CH1)], sem)
            cp0.wait()
            cp1.wait()

            def blk_body(blk, prev):
                rbase = 8 * blk
                hv = half_v[pl.ds(rbase, 16)] * D
                for t in range(8):
                    u = rbase + t
                    off = hv[t]
                    cur = tuple(
                        rows_v[u, pl.ds(off + 16 * j, 16)]
                        for j in range(VECS))
                    for j in range(VECS):
                        outb_v[si, u, pl.ds(16 * j, 16)] = jnp.maximum(
                            prev[j] * w0[j] + cur[j] * w1[j], 0.0)
                    prev = cur
                return prev

            lax.fori_loop(0, U // 8, blk_body, (zero,) * VECS)

        do_seq(0)
        do_seq(1)
        pltpu.sync_copy(outb_v, out_hbm.at[pl.ds(pair0 * 2, 2)])
        return carry

    lax.fori_loop(0, PAIRS_PER_W, pair_body, 0)


_sc_call = pl.kernel(
    _sc_body,
    out_type=jax.ShapeDtypeStruct((N, U, D), jnp.float32),
    mesh=plsc.VectorSubcoreMesh(core_axis_name="c", subcore_axis_name="s"),
    scratch_types=[
        pltpu.VMEM((U,), jnp.int32),
        pltpu.VMEM((U + 16,), jnp.int32),
        pltpu.VMEM((U, 2 * D), jnp.float32),
        pltpu.VMEM((2, U, D), jnp.float32),
        pltpu.VMEM((2, D), jnp.float32),
        pltpu.SemaphoreType.DMA,
    ],
)


@jax.jit
def kernel(y, table, conv_w):
    y_flat = y.reshape(N * U).astype(jnp.int32)
    bh = TB // 2
    y2 = (y_flat // TB) * bh + (y_flat % bh)
    half = (y_flat // bh) & 1
    t2 = _tc_repack(table.T)
    w = conv_w.T
    return _sc_call(y2, half, t2, w)


# speedup vs baseline: 2.7945x; 1.3598x over previous
import jax
import jax.numpy as jnp
from jax import lax
from jax.experimental import pallas as pl
from jax.experimental.pallas import tpu as pltpu
from jax.experimental.pallas import tpu_sc as plsc

N = 1024
U = 200
D = 64
VECS = D // 16

_info = plsc.get_sparse_core_info()
NC, NS = _info.num_cores, _info.num_subcores
NW = NC * NS
SEQ_PER_W = N // NW
PAIRS_PER_W = SEQ_PER_W // 2

CH0 = 128
CH1 = U - CH0

V = 1000000
TB = 8192
TGRID = (V + TB - 1) // TB


def _tc_repack_body(tT_ref, out_ref):
    x = tT_ref[...]
    xt = x.T
    out_ref[...] = jnp.concatenate([xt[0:TB // 2], xt[TB // 2:TB]], axis=1)


_tc_repack = pl.pallas_call(
    _tc_repack_body,
    out_shape=jax.ShapeDtypeStruct((TGRID * (TB // 2), 128), jnp.float32),
    grid=(TGRID,),
    in_specs=[pl.BlockSpec((64, TB), lambda i: (0, i))],
    out_specs=pl.BlockSpec((TB // 2, 128), lambda i: (i, 0)),
)


def _sc_body(y2_hbm, half_hbm, table_hbm, w_hbm, out_hbm,
             idx_v, half_v, rows_v, outb_v, w_v, sem):
    wid = lax.axis_index("s") * NC + lax.axis_index("c")

    pltpu.sync_copy(w_hbm, w_v)
    w0 = [w_v[0, pl.ds(16 * j, 16)] for j in range(VECS)]
    w1 = [w_v[1, pl.ds(16 * j, 16)] for j in range(VECS)]
    zero = jnp.zeros((16,), jnp.float32)

    def pair_body(p_i, carry):
        pair0 = wid * PAIRS_PER_W + p_i
        pair_base = pair0 * 2 * U

        def do_seq(si):
            base = pair_base + si * U
            pltpu.sync_copy(y2_hbm.at[pl.ds(base, U)], idx_v)
            pltpu.sync_copy(half_hbm.at[pl.ds(base, U)],
                            half_v.at[pl.ds(0, U)])
            cp0 = pltpu.async_copy(
                table_hbm.at[idx_v.at[pl.ds(0, CH0)]],
                rows_v.at[pl.ds(0, CH0)], sem)
            cp1 = pltpu.async_copy(
                table_hbm.at[idx_v.at[pl.ds(CH0, CH1)]],
                rows_v.at[pl.ds(CH0, CH1)], sem)
            cp0.wait()
            cp1.wait()

            def blk_body(blk, prev):
                rbase = 8 * blk
                hv = half_v[pl.ds(rbase, 16)] * D
                for t in range(8):
                    u = rbase + t
                    off = hv[t]
                    cur = tuple(
                        rows_v[u, pl.ds(off + 16 * j, 16)]
                        for j in range(VECS))
                    for j in range(VECS):
                        outb_v[si, u, pl.ds(16 * j, 16)] = jnp.maximum(
                            prev[j] * w0[j] + cur[j] * w1[j], 0.0)
                    prev = cur
                return prev

            lax.fori_loop(0, U // 8, blk_body, (zero,) * VECS)

        do_seq(0)
        do_seq(1)
        pltpu.sync_copy(outb_v, out_hbm.at[pl.ds(pair0 * 2, 2)])
        return carry

    lax.fori_loop(0, PAIRS_PER_W, pair_body, 0)


_sc_call = pl.kernel(
    _sc_body,
    out_type=jax.ShapeDtypeStruct((N, U, D), jnp.float32),
    mesh=plsc.VectorSubcoreMesh(core_axis_name="c", subcore_axis_name="s"),
    scratch_types=[
        pltpu.VMEM((U,), jnp.int32),
        pltpu.VMEM((U + 16,), jnp.int32),
        pltpu.VMEM((U, 2 * D), jnp.float32),
        pltpu.VMEM((2, U, D), jnp.float32),
        pltpu.VMEM((2, D), jnp.float32),
        pltpu.SemaphoreType.DMA,
    ],
)


@jax.jit
def kernel(y, table, conv_w):
    y_flat = y.reshape(N * U).astype(jnp.int32)
    bh = TB // 2
    y2 = (y_flat // TB) * bh + (y_flat % bh)
    half = (y_flat // bh) & 1
    t2 = _tc_repack(table.T)
    w = conv_w.T
    return _sc_call(y2, half, t2, w)


# speedup vs baseline: 2.9677x; 1.0620x over previous
import jax
import jax.numpy as jnp
from jax import lax
from jax.experimental import pallas as pl
from jax.experimental.pallas import tpu as pltpu
from jax.experimental.pallas import tpu_sc as plsc

N = 1024
U = 200
D = 64
VECS = D // 16

_info = plsc.get_sparse_core_info()
NC, NS = _info.num_cores, _info.num_subcores
NW = NC * NS
SEQ_PER_W = N // NW
PAIRS_PER_W = SEQ_PER_W // 2

CH0 = 128
CH1 = U - CH0

V = 1000000
TB = 16384
TGRID = (V + TB - 1) // TB


def _tc_repack_body(tT_ref, out_ref):
    x = tT_ref[...]
    xt = x.T
    out_ref[...] = jnp.concatenate([xt[0:TB // 2], xt[TB // 2:TB]], axis=1)


_tc_repack = pl.pallas_call(
    _tc_repack_body,
    out_shape=jax.ShapeDtypeStruct((TGRID * (TB // 2), 128), jnp.float32),
    grid=(TGRID,),
    in_specs=[pl.BlockSpec((64, TB), lambda i: (0, i))],
    out_specs=pl.BlockSpec((TB // 2, 128), lambda i: (i, 0)),
)


def _sc_body(y2_hbm, half_hbm, table_hbm, w_hbm, out_hbm,
             idx_v, half_v, rows_v, outb_v, w_v, sem):
    wid = lax.axis_index("s") * NC + lax.axis_index("c")

    pltpu.sync_copy(w_hbm, w_v)
    w0 = [w_v[0, pl.ds(16 * j, 16)] for j in range(VECS)]
    w1 = [w_v[1, pl.ds(16 * j, 16)] for j in range(VECS)]
    zero = jnp.zeros((16,), jnp.float32)

    def pair_body(p_i, carry):
        pair0 = wid * PAIRS_PER_W + p_i
        pair_base = pair0 * 2 * U

        def do_seq(si):
            base = pair_base + si * U
            pltpu.sync_copy(y2_hbm.at[pl.ds(base, U)], idx_v)
            pltpu.sync_copy(half_hbm.at[pl.ds(base, U)],
                            half_v.at[pl.ds(0, U)])
            cp0 = pltpu.async_copy(
                table_hbm.at[idx_v.at[pl.ds(0, CH0)]],
                rows_v.at[pl.ds(0, CH0)], sem)
            cp1 = pltpu.async_copy(
                table_hbm.at[idx_v.at[pl.ds(CH0, CH1)]],
                rows_v.at[pl.ds(CH0, CH1)], sem)
            cp0.wait()
            cp1.wait()

            def blk_body(blk, prev):
                rbase = 8 * blk
                hv = half_v[pl.ds(rbase, 16)] * D
                for t in range(8):
                    u = rbase + t
                    off = hv[t]
                    cur = tuple(
                        rows_v[u, pl.ds(off + 16 * j, 16)]
                        for j in range(VECS))
                    for j in range(VECS):
                        outb_v[si, u, pl.ds(16 * j, 16)] = jnp.maximum(
                            prev[j] * w0[j] + cur[j] * w1[j], 0.0)
                    prev = cur
                return prev

            lax.fori_loop(0, U // 8, blk_body, (zero,) * VECS)

        do_seq(0)
        do_seq(1)
        pltpu.sync_copy(outb_v, out_hbm.at[pl.ds(pair0 * 2, 2)])
        return carry

    lax.fori_loop(0, PAIRS_PER_W, pair_body, 0)


_sc_call = pl.kernel(
    _sc_body,
    out_type=jax.ShapeDtypeStruct((N, U, D), jnp.float32),
    mesh=plsc.VectorSubcoreMesh(core_axis_name="c", subcore_axis_name="s"),
    scratch_types=[
        pltpu.VMEM((U,), jnp.int32),
        pltpu.VMEM((U + 16,), jnp.int32),
        pltpu.VMEM((U, 2 * D), jnp.float32),
        pltpu.VMEM((2, U, D), jnp.float32),
        pltpu.VMEM((2, D), jnp.float32),
        pltpu.SemaphoreType.DMA,
    ],
)


@jax.jit
def kernel(y, table, conv_w):
    y_flat = y.reshape(N * U).astype(jnp.int32)
    bh = TB // 2
    y2 = (y_flat // TB) * bh + (y_flat % bh)
    half = (y_flat // bh) & 1
    t2 = _tc_repack(table.T)
    w = conv_w.T
    return _sc_call(y2, half, t2, w)


# speedup vs baseline: 3.0676x; 1.0337x over previous
import jax
import jax.numpy as jnp
from jax import lax
from jax.experimental import pallas as pl
from jax.experimental.pallas import tpu as pltpu
from jax.experimental.pallas import tpu_sc as plsc

N = 1024
U = 200
D = 64
VECS = D // 16

_info = plsc.get_sparse_core_info()
NC, NS = _info.num_cores, _info.num_subcores
NW = NC * NS
SEQ_PER_W = N // NW
PAIRS_PER_W = SEQ_PER_W // 2

CH0 = 128
CH1 = U - CH0

V = 1000000
TB = 32768
TGRID = (V + TB - 1) // TB


def _tc_repack_body(tT_ref, out_ref):
    x = tT_ref[...]
    xt = x.T
    out_ref[...] = jnp.concatenate([xt[0:TB // 2], xt[TB // 2:TB]], axis=1)


_tc_repack = pl.pallas_call(
    _tc_repack_body,
    out_shape=jax.ShapeDtypeStruct((TGRID * (TB // 2), 128), jnp.float32),
    grid=(TGRID,),
    in_specs=[pl.BlockSpec((64, TB), lambda i: (0, i))],
    out_specs=pl.BlockSpec((TB // 2, 128), lambda i: (i, 0)),
)


def _sc_body(y2_hbm, half_hbm, table_hbm, w_hbm, out_hbm,
             idx_v, half_v, rows_v, outb_v, w_v, sem):
    wid = lax.axis_index("s") * NC + lax.axis_index("c")

    pltpu.sync_copy(w_hbm, w_v)
    w0 = [w_v[0, pl.ds(16 * j, 16)] for j in range(VECS)]
    w1 = [w_v[1, pl.ds(16 * j, 16)] for j in range(VECS)]
    zero = jnp.zeros((16,), jnp.float32)

    def pair_body(p_i, carry):
        pair0 = wid * PAIRS_PER_W + p_i
        pair_base = pair0 * 2 * U

        def do_seq(si):
            base = pair_base + si * U
            pltpu.sync_copy(y2_hbm.at[pl.ds(base, U)], idx_v)
            pltpu.sync_copy(half_hbm.at[pl.ds(base, U)],
                            half_v.at[pl.ds(0, U)])
            cp0 = pltpu.async_copy(
                table_hbm.at[idx_v.at[pl.ds(0, CH0)]],
                rows_v.at[pl.ds(0, CH0)], sem)
            cp1 = pltpu.async_copy(
                table_hbm.at[idx_v.at[pl.ds(CH0, CH1)]],
                rows_v.at[pl.ds(CH0, CH1)], sem)
            cp0.wait()
            cp1.wait()

            def blk_body(blk, prev):
                rbase = 8 * blk
                hv = half_v[pl.ds(rbase, 16)] * D
                for t in range(8):
                    u = rbase + t
                    off = hv[t]
                    cur = tuple(
                        rows_v[u, pl.ds(off + 16 * j, 16)]
                        for j in range(VECS))
                    for j in range(VECS):
                        outb_v[si, u, pl.ds(16 * j, 16)] = jnp.maximum(
                            prev[j] * w0[j] + cur[j] * w1[j], 0.0)
                    prev = cur
                return prev

            lax.fori_loop(0, U // 8, blk_body, (zero,) * VECS)

        do_seq(0)
        do_seq(1)
        pltpu.sync_copy(outb_v, out_hbm.at[pl.ds(pair0 * 2, 2)])
        return carry

    lax.fori_loop(0, PAIRS_PER_W, pair_body, 0)


_sc_call = pl.kernel(
    _sc_body,
    out_type=jax.ShapeDtypeStruct((N, U, D), jnp.float32),
    mesh=plsc.VectorSubcoreMesh(core_axis_name="c", subcore_axis_name="s"),
    scratch_types=[
        pltpu.VMEM((U,), jnp.int32),
        pltpu.VMEM((U + 16,), jnp.int32),
        pltpu.VMEM((U, 2 * D), jnp.float32),
        pltpu.VMEM((2, U, D), jnp.float32),
        pltpu.VMEM((2, D), jnp.float32),
        pltpu.SemaphoreType.DMA,
    ],
)


@jax.jit
def kernel(y, table, conv_w):
    y_flat = y.reshape(N * U).astype(jnp.int32)
    bh = TB // 2
    y2 = (y_flat // TB) * bh + (y_flat % bh)
    half = (y_flat // bh) & 1
    t2 = _tc_repack(table.T)
    w = conv_w.T
    return _sc_call(y2, half, t2, w)


# speedup vs baseline: 3.6228x; 1.1810x over previous
import jax
import jax.numpy as jnp
from jax import lax
from jax.experimental import pallas as pl
from jax.experimental.pallas import tpu as pltpu
from jax.experimental.pallas import tpu_sc as plsc

N = 1024
U = 200
D = 64
VECS = D // 16

_info = plsc.get_sparse_core_info()
NC, NS = _info.num_cores, _info.num_subcores
NW = NC * NS
SEQ_PER_W = N // NW
PAIRS_PER_W = SEQ_PER_W // 2

CH0 = 128
CH1 = U - CH0
HB = U + 24

V = 1000000
TB = 32768
TGRID = (V + TB - 1) // TB


def _tc_repack_body(tT_ref, out_ref):
    x = tT_ref[...]
    xt = x.T
    out_ref[...] = jnp.concatenate([xt[0:TB // 2], xt[TB // 2:TB]], axis=1)


_tc_repack = pl.pallas_call(
    _tc_repack_body,
    out_shape=jax.ShapeDtypeStruct((TGRID * (TB // 2), 128), jnp.float32),
    grid=(TGRID,),
    in_specs=[pl.BlockSpec((64, TB), lambda i: (0, i))],
    out_specs=pl.BlockSpec((TB // 2, 128), lambda i: (i, 0)),
)


def _sc_body(y2_hbm, half_hbm, table_hbm, w_hbm, out_hbm,
             idx_v, half_v, rows_v, outb_v, w_v,
             gsem0, gsem1, osem0, osem1):
    wid = lax.axis_index("s") * NC + lax.axis_index("c")
    gsems = (gsem0, gsem1)
    osems = (osem0, osem1)

    pltpu.sync_copy(w_hbm, w_v)
    w0 = [w_v[0, pl.ds(16 * j, 16)] for j in range(VECS)]
    w1 = [w_v[1, pl.ds(16 * j, 16)] for j in range(VECS)]
    zero = jnp.zeros((16,), jnp.float32)

    def gather_copies(b):
        return (
            pltpu.make_async_copy(
                table_hbm.at[idx_v.at[pl.ds(b * U, CH0)]],
                rows_v.at[b, pl.ds(0, CH0)], gsems[b]),
            pltpu.make_async_copy(
                table_hbm.at[idx_v.at[pl.ds(b * U + CH0, CH1)]],
                rows_v.at[b, pl.ds(CH0, CH1)], gsems[b]),
        )

    def gather_start(s_dyn, b):
        base = (wid * SEQ_PER_W + s_dyn) * U
        pltpu.sync_copy(y2_hbm.at[pl.ds(base, U)],
                        idx_v.at[pl.ds(b * U, U)])
        pltpu.sync_copy(half_hbm.at[pl.ds(base, U)],
                        half_v.at[pl.ds(b * HB, U)])
        for cp in gather_copies(b):
            cp.start()

    def gather_wait(b):
        for cp in gather_copies(b):
            cp.wait()

    def wb_copy(s_dyn, b):
        return pltpu.make_async_copy(
            outb_v.at[b],
            out_hbm.at[wid * SEQ_PER_W + s_dyn],
            osems[b])

    def compute(b):
        def blk_body(blk, prev):
            rbase = 8 * blk
            hv = half_v[pl.ds(b * HB + rbase, 16)] * D
            for t in range(8):
                u = rbase + t
                off = hv[t]
                cur = tuple(
                    rows_v[b, u, pl.ds(off + 16 * j, 16)]
                    for j in range(VECS))
                for j in range(VECS):
                    outb_v[b, u, pl.ds(16 * j, 16)] = jnp.maximum(
                        prev[j] * w0[j] + cur[j] * w1[j], 0.0)
                prev = cur
            return prev

        lax.fori_loop(0, U // 8, blk_body, (zero,) * VECS)

    gather_start(0, 0)

    def super_body(i, carry):
        for b in (0, 1):
            s = 2 * i + b

            @pl.when(s < SEQ_PER_W - 1)
            def _():
                gather_start(s + 1, 1 - b)

            gather_wait(b)

            @pl.when(s >= 2)
            def _():
                wb_copy(s - 2, b).wait()

            compute(b)
            wb_copy(s, b).start()
        return carry

    lax.fori_loop(0, SEQ_PER_W // 2, super_body, 0)
    wb_copy(SEQ_PER_W - 2, 0).wait()
    wb_copy(SEQ_PER_W - 1, 1).wait()


_sc_call = pl.kernel(
    _sc_body,
    out_type=jax.ShapeDtypeStruct((N, U, D), jnp.float32),
    mesh=plsc.VectorSubcoreMesh(core_axis_name="c", subcore_axis_name="s"),
    scratch_types=[
        pltpu.VMEM((2 * U,), jnp.int32),
        pltpu.VMEM((2 * HB,), jnp.int32),
        pltpu.VMEM((2, U, 2 * D), jnp.float32),
        pltpu.VMEM((2, U, D), jnp.float32),
        pltpu.VMEM((2, D), jnp.float32),
        pltpu.SemaphoreType.DMA,
        pltpu.SemaphoreType.DMA,
        pltpu.SemaphoreType.DMA,
        pltpu.SemaphoreType.DMA,
    ],
)


@jax.jit
def kernel(y, table, conv_w):
    y_flat = y.reshape(N * U).astype(jnp.int32)
    bh = TB // 2
    y2 = (y_flat // TB) * bh + (y_flat % bh)
    half = (y_flat // bh) & 1
    t2 = _tc_repack(table.T)
    w = conv_w.T
    return _sc_call(y2, half, t2, w)
